# baseline (speedup 1.0000x reference)
import functools

import jax
import jax.numpy as jnp
from jax import lax
from jax.experimental import pallas as pl
from jax.experimental.pallas import tpu as pltpu
from jax.experimental.pallas import tpu_sc as plsc

D = 64
L = 16
NC = 2
NS = 16
NW = NC * NS


def _rsqrt_vec(x):
    i = lax.bitcast_convert_type(x, jnp.int32)
    i = jnp.int32(0x5F3759DF) - (i >> 1)
    y = lax.bitcast_convert_type(i, jnp.float32)
    half = x * jnp.float32(0.5)
    for _ in range(3):
        y = y * (jnp.float32(1.5) - half * y * y)
    return y


def _make_kernel(batch):
    assert batch % (8 * NW) == 0
    b_per_w = batch // NW
    mesh = plsc.VectorSubcoreMesh(
        core_axis_name="c", subcore_axis_name="s",
        num_cores=NC, num_subcores=NS,
    )

    @functools.partial(
        pl.kernel,
        out_type=jax.ShapeDtypeStruct((batch, D), jnp.float32),
        mesh=mesh,
        scratch_types=[
            pltpu.VMEM((b_per_w,), jnp.int32),
            pltpu.VMEM((b_per_w, D), jnp.float32),
            pltpu.SemaphoreType.DMA,
        ],
        compiler_params=pltpu.CompilerParams(use_tc_tiling_on_sc=False),
    )
    def body(x_hbm, table_hbm, out_hbm, idx_v, rows_v, sem):
        wid = lax.axis_index("s") * NC + lax.axis_index("c")
        base = wid * b_per_w
        pltpu.sync_copy(x_hbm.at[pl.ds(base, b_per_w)], idx_v)
        pltpu.async_copy(table_hbm.at[idx_v], rows_v, sem).wait()

        def row_fn(r, carry):
            v0 = rows_v[r, pl.ds(0, L)]
            v1 = rows_v[r, pl.ds(L, L)]
            v2 = rows_v[r, pl.ds(2 * L, L)]
            v3 = rows_v[r, pl.ds(3 * L, L)]
            s = v0 * v0 + v1 * v1 + v2 * v2 + v3 * v3
            iot = lax.iota(jnp.int32, L)
            for k in (8, 4, 2, 1):
                perm = iot ^ k
                s = s + jnp.take_along_axis(
                    s, perm, axis=0, mode="promise_in_bounds")
            rs = jnp.minimum(_rsqrt_vec(s), jnp.float32(1e12))
            rows_v[r, pl.ds(0, L)] = v0 * rs
            rows_v[r, pl.ds(L, L)] = v1 * rs
            rows_v[r, pl.ds(2 * L, L)] = v2 * rs
            rows_v[r, pl.ds(3 * L, L)] = v3 * rs
            return carry

        lax.fori_loop(0, b_per_w, row_fn, 0)
        pltpu.sync_copy(rows_v, out_hbm.at[pl.ds(base, b_per_w)])

    return body


def kernel(X, table):
    batch = X.shape[0]
    return _make_kernel(batch)(X.astype(jnp.int32), table)
